# baseline (speedup 1.0000x reference)
import jax
import jax.numpy as jnp
from jax.experimental import pallas as pl
from jax.experimental.pallas import tpu as pltpu

_BC = 8


def _body(x_ref, a1_ref, a2_ref, w_ref, b_ref, o_ref, a1sq_ref, a2sq_ref):
    f32 = jnp.float32
    i = pl.program_id(0)

    @pl.when(i == 0)
    def _prep():
        a1 = a1_ref[:]
        a2 = a2_ref[:]
        a1sq_ref[:] = jnp.dot(a1, a1, preferred_element_type=f32)
        a2sq_ref[:] = jnp.dot(a2, a2, preferred_element_type=f32)

    xb = x_ref[:]
    u1 = jnp.dot(xb, a1_ref[:], preferred_element_type=f32)
    u2 = jnp.dot(xb, a1sq_ref[:], preferred_element_type=f32)
    v1 = jnp.dot(xb, a2_ref[:], preferred_element_type=f32)
    v2 = jnp.dot(xb, a2sq_ref[:], preferred_element_type=f32)

    w = w_ref[:]
    bias = b_ref[:]
    outs = []
    for j in range(_BC):
        sl = slice(j * 64, (j + 1) * 64)
        h = jnp.concatenate([xb[sl], u1[sl], u2[sl], v1[sl], v2[sl]],
                            axis=0)
        outs.append(jnp.dot(w, h, preferred_element_type=f32) + bias)
    o_ref[:] = jnp.concatenate(outs, axis=0)


def kernel(x, adjs, W, b):
    B, C, N = x.shape
    xf = x.reshape(B * C, N)
    a1 = adjs[0]
    a2 = adjs[1]
    b2d = jnp.broadcast_to(b[:, None], (C, N)).astype(jnp.float32)

    rows = _BC * C
    grid = (B // _BC,)
    out = pl.pallas_call(
        _body,
        grid=grid,
        in_specs=[
            pl.BlockSpec((rows, N), lambda i: (i, 0)),
            pl.BlockSpec((N, N), lambda i: (0, 0)),
            pl.BlockSpec((N, N), lambda i: (0, 0)),
            pl.BlockSpec((C, 5 * C), lambda i: (0, 0)),
            pl.BlockSpec((C, N), lambda i: (0, 0)),
        ],
        out_specs=pl.BlockSpec((rows, N), lambda i: (i, 0)),
        out_shape=jax.ShapeDtypeStruct((B * C, N), jnp.float32),
        scratch_shapes=[
            pltpu.VMEM((N, N), jnp.float32),
            pltpu.VMEM((N, N), jnp.float32),
        ],
        compiler_params=pltpu.CompilerParams(
            dimension_semantics=("arbitrary",),
        ),
    )(xf, a1, a2, W, b2d)
    return out.reshape(B, C, N)


# speedup vs baseline: 1.0193x; 1.0193x over previous
import jax
import jax.numpy as jnp
from jax.experimental import pallas as pl
from jax.experimental.pallas import tpu as pltpu

_BC = 8


def _body(x_ref, a1_ref, a2_ref, w_ref, b_ref, o_ref, a1sq_ref, a2sq_ref):
    f32 = jnp.float32
    bf16 = jnp.bfloat16
    i = pl.program_id(0)

    @pl.when(i == 0)
    def _prep():
        a1 = a1_ref[:]
        a2 = a2_ref[:]
        a1sq_ref[:] = jnp.dot(a1, a1, preferred_element_type=f32).astype(bf16)
        a2sq_ref[:] = jnp.dot(a2, a2, preferred_element_type=f32).astype(bf16)

    xb = x_ref[:].astype(bf16)
    u1 = jnp.dot(xb, a1_ref[:], preferred_element_type=f32).astype(bf16)
    u2 = jnp.dot(xb, a1sq_ref[:], preferred_element_type=f32).astype(bf16)
    v1 = jnp.dot(xb, a2_ref[:], preferred_element_type=f32).astype(bf16)
    v2 = jnp.dot(xb, a2sq_ref[:], preferred_element_type=f32).astype(bf16)

    w = w_ref[:]
    bias = b_ref[:]
    outs = []
    for j in range(_BC):
        sl = slice(j * 64, (j + 1) * 64)
        h = jnp.concatenate([xb[sl], u1[sl], u2[sl], v1[sl], v2[sl]],
                            axis=0)
        outs.append(jnp.dot(w, h, preferred_element_type=f32) + bias)
    o_ref[:] = jnp.concatenate(outs, axis=0)


def kernel(x, adjs, W, b):
    B, C, N = x.shape
    xf = x.reshape(B * C, N)
    a1 = adjs[0].astype(jnp.bfloat16)
    a2 = adjs[1].astype(jnp.bfloat16)
    wb = W.astype(jnp.bfloat16)
    b2d = jnp.broadcast_to(b[:, None], (C, N)).astype(jnp.float32)

    rows = _BC * C
    grid = (B // _BC,)
    out = pl.pallas_call(
        _body,
        grid=grid,
        in_specs=[
            pl.BlockSpec((rows, N), lambda i: (i, 0)),
            pl.BlockSpec((N, N), lambda i: (0, 0)),
            pl.BlockSpec((N, N), lambda i: (0, 0)),
            pl.BlockSpec((C, 5 * C), lambda i: (0, 0)),
            pl.BlockSpec((C, N), lambda i: (0, 0)),
        ],
        out_specs=pl.BlockSpec((rows, N), lambda i: (i, 0)),
        out_shape=jax.ShapeDtypeStruct((B * C, N), jnp.float32),
        scratch_shapes=[
            pltpu.VMEM((N, N), jnp.bfloat16),
            pltpu.VMEM((N, N), jnp.bfloat16),
        ],
        compiler_params=pltpu.CompilerParams(
            dimension_semantics=("arbitrary",),
        ),
    )(xf, a1, a2, wb, b2d)
    return out.reshape(B, C, N)


# speedup vs baseline: 1.0961x; 1.0753x over previous
import jax
import jax.numpy as jnp
from jax.experimental import pallas as pl
from jax.experimental.pallas import tpu as pltpu

_BC = 8


def _body(x_ref, a1_ref, a2_ref, w_ref, b_ref, o_ref):
    f32 = jnp.float32
    bf16 = jnp.bfloat16
    xb = x_ref[:].astype(bf16)
    a1 = a1_ref[:]
    a2 = a2_ref[:]
    w = w_ref[:]
    bias = b_ref[:]

    u1b = jnp.dot(xb, a1, preferred_element_type=f32).astype(bf16)
    v1b = jnp.dot(xb, a2, preferred_element_type=f32).astype(bf16)
    u2b = jnp.dot(u1b, a1, preferred_element_type=f32).astype(bf16)
    v2b = jnp.dot(v1b, a2, preferred_element_type=f32).astype(bf16)

    outs = []
    for j in range(_BC):
        sl = slice(j * 64, (j + 1) * 64)
        h = jnp.concatenate([xb[sl], u1b[sl], u2b[sl], v1b[sl], v2b[sl]],
                            axis=0)
        outs.append(jnp.dot(w, h, preferred_element_type=f32) + bias)
    o_ref[:] = jnp.concatenate(outs, axis=0)


def kernel(x, adjs, W, b):
    B, C, N = x.shape
    xf = x.reshape(B * C, N)
    a1 = adjs[0].astype(jnp.bfloat16)
    a2 = adjs[1].astype(jnp.bfloat16)
    wb = W.astype(jnp.bfloat16)
    b2d = jnp.broadcast_to(b[:, None], (C, N)).astype(jnp.float32)

    rows = _BC * C
    grid = (B // _BC,)
    out = pl.pallas_call(
        _body,
        grid=grid,
        in_specs=[
            pl.BlockSpec((rows, N), lambda i: (i, 0)),
            pl.BlockSpec((N, N), lambda i: (0, 0)),
            pl.BlockSpec((N, N), lambda i: (0, 0)),
            pl.BlockSpec((C, 5 * C), lambda i: (0, 0)),
            pl.BlockSpec((C, N), lambda i: (0, 0)),
        ],
        out_specs=pl.BlockSpec((rows, N), lambda i: (i, 0)),
        out_shape=jax.ShapeDtypeStruct((B * C, N), jnp.float32),
        compiler_params=pltpu.CompilerParams(
            dimension_semantics=("arbitrary",),
        ),
    )(xf, a1, a2, wb, b2d)
    return out.reshape(B, C, N)
